# Initial kernel scaffold; baseline (speedup 1.0000x reference)
import jax
import jax.numpy as jnp
from jax.experimental import pallas as pl


def kernel(x_token_idxs, edge_index, batch, token_emb, W_proj, b_proj, W_g1, b_g1, W_g2, b_g2, W_fc1, b_fc1, W_out, b_out):
    raise NotImplementedError("write your pallas kernel here")


# speedup vs baseline: 7.8442x; 7.8442x over previous
import jax
import jax.numpy as jnp
from jax import lax
from jax.experimental import pallas as pl
from jax.experimental.pallas import tpu as pltpu
from jax.experimental.pallas import tpu_sc as plsc

N = 50000
E = 800000
T = 20
V = 100000
D_TOK = 64
H = 128
G = 512

NC = 2
NS = 16
NW = NC * NS

NPW = 1664
N_PAD = NW * NPW
E_PAD = 819200
EPW = E_PAD // NW
QE = E_PAD // NS
G_PAD = 1024

CE = 6400
ECH = QE // CE
NSL = 16
CD = 5120
DCH = EPW // CD

BN = 512
NB = N_PAD // BN

_mesh = plsc.VectorSubcoreMesh(core_axis_name="c", subcore_axis_name="s",
                               num_cores=NC, num_subcores=NS)
_sc_params = pltpu.CompilerParams(use_tc_tiling_on_sc=False)


def _wid():
    return lax.axis_index("s") * NC + lax.axis_index("c")


def _emb_body(idx_hbm, table_hbm, out_hbm, idx_v, rows_v, sum_v, sem):
    wid = _wid()

    def chunk(ch, _):
        nbase = wid * NPW + ch * 128

        def zero(i, _):
            for r in range(4):
                sum_v[i, pl.ds(r * 16, 16)] = jnp.zeros((16,), jnp.float32)
            return 0

        lax.fori_loop(0, 128, zero, 0)
        for k in range(T):
            pltpu.sync_copy(
                idx_hbm.at[pl.ds((wid * T + k) * NPW + ch * 128, 128)], idx_v)
            pltpu.async_copy(table_hbm.at[idx_v], rows_v, sem).wait()

            def add(i, _):
                for r in range(4):
                    sl = pl.ds(r * 16, 16)
                    sum_v[i, sl] = sum_v[i, sl] + rows_v[i, sl]
                return 0

            lax.fori_loop(0, 128, add, 0)
        pltpu.sync_copy(sum_v, out_hbm.at[pl.ds(nbase, 128)])
        return 0

    lax.fori_loop(0, NPW // 128, chunk, 0)


def _emb_call(idx_tm, table):
    f = pl.kernel(
        _emb_body,
        out_type=jax.ShapeDtypeStruct((N_PAD, 64), jnp.float32),
        mesh=_mesh,
        compiler_params=_sc_params,
        scratch_types=[
            pltpu.VMEM((128,), jnp.int32),
            pltpu.VMEM((128, 64), jnp.float32),
            pltpu.VMEM((128, 64), jnp.float32),
            pltpu.SemaphoreType.DMA,
        ],
    )
    return f(idx_tm, table)


def _deg_body(dst_hbm, batch_hbm, deg_out, cnt_out,
              dst_v, bidx_v, ones_v, deg_sh, cnt_sh, zb):
    cid = lax.axis_index("c")
    sid = lax.axis_index("s")
    wid = sid * NC + cid
    stride = N_PAD // NS

    def fill0(i, _):
        zb[pl.ds(i * 16, 16)] = jnp.zeros((16,), jnp.float32)
        return 0

    lax.fori_loop(0, stride // 16, fill0, 0)

    def fill1(i, _):
        ones_v[pl.ds(i * 16, 16)] = jnp.ones((16,), jnp.float32)
        return 0

    lax.fori_loop(0, CD // 16, fill1, 0)

    pltpu.sync_copy(zb, deg_sh.at[pl.ds(sid * stride, stride)])
    pltpu.sync_copy(zb.at[pl.ds(0, 64)], cnt_sh.at[pl.ds(sid * 64, 64)])
    plsc.subcore_barrier()

    def dchunk(ch, _):
        ebase = wid * EPW + ch * CD
        pltpu.sync_copy(dst_hbm.at[pl.ds(ebase, CD)], dst_v)
        pltpu.sync_copy(ones_v, deg_sh.at[dst_v], add=True)
        return 0

    lax.fori_loop(0, DCH, dchunk, 0)

    pltpu.sync_copy(batch_hbm.at[pl.ds(wid * NPW, NPW)], bidx_v)
    pltpu.sync_copy(ones_v.at[pl.ds(0, NPW)], cnt_sh.at[bidx_v], add=True)

    plsc.subcore_barrier()
    pltpu.sync_copy(deg_sh.at[pl.ds(sid * stride, stride)], zb)
    pltpu.sync_copy(zb, deg_out.at[pl.ds(cid * N_PAD + sid * stride, stride)])
    pltpu.sync_copy(cnt_sh.at[pl.ds(sid * 64, 64)], zb.at[pl.ds(0, 64)])
    pltpu.sync_copy(zb.at[pl.ds(0, 64)],
                    cnt_out.at[pl.ds(cid * G_PAD + sid * 64, 64)])


def _deg_call(dst_flat, batch_flat):
    f = pl.kernel(
        _deg_body,
        out_type=[
            jax.ShapeDtypeStruct((2 * N_PAD,), jnp.float32),
            jax.ShapeDtypeStruct((2 * G_PAD,), jnp.float32),
        ],
        mesh=_mesh,
        compiler_params=_sc_params,
        scratch_types=[
            pltpu.VMEM((CD,), jnp.int32),
            pltpu.VMEM((NPW,), jnp.int32),
            pltpu.VMEM((CD,), jnp.float32),
            pltpu.VMEM_SHARED((N_PAD,), jnp.float32),
            pltpu.VMEM_SHARED((G_PAD,), jnp.float32),
            pltpu.VMEM((N_PAD // NS,), jnp.float32),
        ],
    )
    return f(dst_flat, batch_flat)


def _scat_body(y_hbm, src_hbm, dst_hbm, zeros_hbm, acc_out,
               src_v, dst_v, rows_v, zz, bb, acc_sh, sem):
    cid = lax.axis_index("c")
    sid = lax.axis_index("s")
    stride = N_PAD // NS

    pltpu.sync_copy(zeros_hbm, zz)

    for q in range(NSL // NC):
        s = cid * (NSL // NC) + q
        for w in range(4):
            pltpu.sync_copy(zz, acc_sh.at[pl.ds(sid * stride + w * 832, 832)])
        plsc.subcore_barrier()

        def chunk(ch, _):
            ebase = sid * QE + ch * CE
            pltpu.sync_copy(src_hbm.at[pl.ds(ebase, CE)], src_v)
            pltpu.sync_copy(dst_hbm.at[pl.ds(ebase, CE)], dst_v)

            def mkidx(i, _):
                sl = pl.ds(i * 16, 16)
                src_v[sl] = src_v[sl] * NSL + s
                return 0

            lax.fori_loop(0, CE // 16, mkidx, 0)
            pltpu.async_copy(y_hbm.at[src_v], rows_v, sem).wait()
            pltpu.sync_copy(rows_v, acc_sh.at[dst_v], add=True)
            return 0

        lax.fori_loop(0, ECH, chunk, 0)
        plsc.subcore_barrier()
        for w in range(4):
            base = sid * stride + w * 832
            pltpu.sync_copy(acc_sh.at[pl.ds(base, 832)], bb)
            pltpu.sync_copy(bb, acc_out.at[pl.ds(base, 832), s])
        plsc.subcore_barrier()


def _scat_call(y_flat, src_flat, dst_flat, zeros832):
    f = pl.kernel(
        _scat_body,
        out_type=jax.ShapeDtypeStruct((N_PAD, NSL, 8), jnp.float32),
        mesh=_mesh,
        compiler_params=_sc_params,
        scratch_types=[
            pltpu.VMEM((CE,), jnp.int32),
            pltpu.VMEM((CE,), jnp.int32),
            pltpu.VMEM((CE, 8), jnp.float32),
            pltpu.VMEM((832, 8), jnp.float32),
            pltpu.VMEM((832, 8), jnp.float32),
            pltpu.VMEM_SHARED((N_PAD, 8), jnp.float32),
            pltpu.SemaphoreType.DMA,
        ],
    )
    return f(y_flat, src_flat, dst_flat, zeros832)


def _proj_body(ts_ref, deg_ref, wp_ref, bp_ref, wg1_ref, bg1_ref,
               y_ref, dinv8):
    ts = ts_ref[...] * (1.0 / T)
    z = jnp.dot(ts, wp_ref[...], preferred_element_type=jnp.float32) + bp_ref[...]
    z = jnp.dot(z, wg1_ref[...], preferred_element_type=jnp.float32) + bg1_ref[...]
    deg = lax.dot_general(deg_ref[...], jnp.ones((2, 1), jnp.float32),
                          (((0,), (0,)), ((), ()))) + 1.0
    dinv = lax.rsqrt(deg)
    y_ref[...] = z * dinv
    dinv8[...] = jnp.broadcast_to(dinv, (BN, 8))


def _proj_call(ts, deg2, w_proj, b_proj, w_g1, b_g1):
    return pl.pallas_call(
        _proj_body,
        grid=(NB,),
        in_specs=[
            pl.BlockSpec((BN, 64), lambda i: (i, 0)),
            pl.BlockSpec((2, BN), lambda i: (0, i)),
            pl.BlockSpec((64, 64), lambda i: (0, 0)),
            pl.BlockSpec((1, 64), lambda i: (0, 0)),
            pl.BlockSpec((64, 128), lambda i: (0, 0)),
            pl.BlockSpec((1, 128), lambda i: (0, 0)),
        ],
        out_specs=[pl.BlockSpec((BN, 128), lambda i: (i, 0)),
                   pl.BlockSpec((BN, 8), lambda i: (i, 0))],
        out_shape=[jax.ShapeDtypeStruct((N_PAD, 128), jnp.float32),
                   jax.ShapeDtypeStruct((N_PAD, 8), jnp.float32)],
    )(ts, deg2, w_proj, b_proj, w_g1, b_g1)


def _mid_body(acc_ref, y_ref, dinv8, bg1_ref, wg2_ref, bg2_ref, o_ref):
    dinv = dinv8[...][:, 0:1]
    x2 = jnp.maximum(dinv * (acc_ref[...] + y_ref[...]) + bg1_ref[...], 0.0)
    z2 = jnp.dot(x2, wg2_ref[...], preferred_element_type=jnp.float32) + bg2_ref[...]
    o_ref[...] = z2 * dinv


def _mid_call(agg1, y1, dinv8, b_g1, w_g2, b_g2):
    return pl.pallas_call(
        _mid_body,
        grid=(NB,),
        in_specs=[
            pl.BlockSpec((BN, 128), lambda i: (i, 0)),
            pl.BlockSpec((BN, 128), lambda i: (i, 0)),
            pl.BlockSpec((BN, 8), lambda i: (i, 0)),
            pl.BlockSpec((1, 128), lambda i: (0, 0)),
            pl.BlockSpec((128, 128), lambda i: (0, 0)),
            pl.BlockSpec((1, 128), lambda i: (0, 0)),
        ],
        out_specs=pl.BlockSpec((BN, 128), lambda i: (i, 0)),
        out_shape=jax.ShapeDtypeStruct((N_PAD, 128), jnp.float32),
    )(agg1, y1, dinv8, b_g1, w_g2, b_g2)


def _relu_pool_body(acc_ref, y_ref, dinv8, bg2_ref, batch_ref, pool_ref):
    i = pl.program_id(0)
    dinv = dinv8[...][:, 0:1]
    x3 = jnp.maximum(
        dinv * (acc_ref[...] + y_ref[...]) + bg2_ref[...], 0.0)
    b = batch_ref[...][:, 0:1]
    gids = lax.broadcasted_iota(jnp.int32, (1, G), 1)
    onehot = (b == gids).astype(jnp.float32)
    part = lax.dot_general(onehot, x3, (((0,), (0,)), ((), ())),
                           preferred_element_type=jnp.float32)

    @pl.when(i == 0)
    def _():
        pool_ref[...] = jnp.zeros_like(pool_ref)

    pool_ref[...] += part


def _relu_pool_call(agg2, y2, dinv8, b_g2, batch8):
    return pl.pallas_call(
        _relu_pool_body,
        grid=(NB,),
        in_specs=[
            pl.BlockSpec((BN, 128), lambda i: (i, 0)),
            pl.BlockSpec((BN, 128), lambda i: (i, 0)),
            pl.BlockSpec((BN, 8), lambda i: (i, 0)),
            pl.BlockSpec((1, 128), lambda i: (0, 0)),
            pl.BlockSpec((BN, 8), lambda i: (i, 0)),
        ],
        out_specs=pl.BlockSpec((G, 128), lambda i: (0, 0)),
        out_shape=jax.ShapeDtypeStruct((G, 128), jnp.float32),
    )(agg2, y2, dinv8, b_g2, batch8)


def _head_body(pool_ref, cnt_ref, wfc_ref, bfc_ref, wout_ref, bout_ref,
               probs_ref, logits_ref):
    cnt = lax.dot_general(cnt_ref[...], jnp.ones((2, 1), jnp.float32),
                          (((0,), (0,)), ((), ())))
    g = pool_ref[...] / jnp.maximum(cnt[:G], 1.0)
    h = jnp.maximum(
        jnp.dot(g, wfc_ref[...], preferred_element_type=jnp.float32)
        + bfc_ref[...], 0.0)
    lo = jnp.dot(h, wout_ref[...], preferred_element_type=jnp.float32) + bout_ref[...]
    probs_ref[...] = jax.nn.sigmoid(lo)
    logits_ref[...] = lo


def _head_call(pool, cnt2, w_fc1, b_fc1, w_out8, b_out8):
    return pl.pallas_call(
        _head_body,
        grid=(1,),
        in_specs=[
            pl.BlockSpec((G, 128), lambda i: (0, 0)),
            pl.BlockSpec((2, G_PAD), lambda i: (0, 0)),
            pl.BlockSpec((128, 128), lambda i: (0, 0)),
            pl.BlockSpec((1, 128), lambda i: (0, 0)),
            pl.BlockSpec((128, 8), lambda i: (0, 0)),
            pl.BlockSpec((1, 8), lambda i: (0, 0)),
        ],
        out_specs=[pl.BlockSpec((G, 8), lambda i: (0, 0))] * 2,
        out_shape=[jax.ShapeDtypeStruct((G, 8), jnp.float32)] * 2,
    )(pool, cnt2, w_fc1, b_fc1, w_out8, b_out8)


def kernel(x_token_idxs, edge_index, batch, token_emb, W_proj, b_proj,
           W_g1, b_g1, W_g2, b_g2, W_fc1, b_fc1, W_out, b_out):
    f32 = jnp.float32
    i32 = jnp.int32

    pad_n = N_PAD - N
    pad_e = E_PAD - E
    tok_pad = (jnp.arange(pad_n, dtype=i32)[:, None] * T
               + jnp.arange(T, dtype=i32)[None, :]) % V
    idx_nt = jnp.concatenate([x_token_idxs.astype(i32), tok_pad], axis=0)
    idx_tm = idx_nt.reshape(NW, NPW, T).transpose(0, 2, 1).reshape(-1)
    table = token_emb.astype(f32)

    src_flat = jnp.concatenate(
        [edge_index[0].astype(i32), jnp.arange(pad_e, dtype=i32) % N])
    dst_flat = jnp.concatenate(
        [edge_index[1].astype(i32), N + (jnp.arange(pad_e, dtype=i32) % pad_n)])
    batch_flat = jnp.concatenate(
        [batch.astype(i32), G + (jnp.arange(pad_n, dtype=i32) % 512)])

    zeros832 = jnp.zeros((832, 8), f32)
    b_proj_r = b_proj.reshape(1, 64).astype(f32)
    b_g1_r = b_g1.reshape(1, 128).astype(f32)
    b_g2_r = b_g2.reshape(1, 128).astype(f32)
    b_fc1_r = b_fc1.reshape(1, 128).astype(f32)
    w_out8 = jnp.pad(W_out.astype(f32), ((0, 0), (0, 7)))
    b_out8 = jnp.pad(b_out.astype(f32), (0, 7)).reshape(1, 8)

    ts = _emb_call(idx_tm, table)
    degf, cntf = _deg_call(dst_flat, batch_flat)
    deg2 = degf.reshape(2, N_PAD)
    cnt2 = cntf.reshape(2, G_PAD)

    y1, dinv8 = _proj_call(ts, deg2, W_proj.astype(f32), b_proj_r,
                           W_g1.astype(f32), b_g1_r)

    acc1 = _scat_call(y1.reshape(NSL * N_PAD, 8), src_flat, dst_flat, zeros832)
    y2 = _mid_call(acc1.reshape(N_PAD, 128), y1, dinv8, b_g1_r,
                   W_g2.astype(f32), b_g2_r)

    acc2 = _scat_call(y2.reshape(NSL * N_PAD, 8), src_flat, dst_flat, zeros832)
    batch8 = jnp.broadcast_to(batch_flat[:, None], (N_PAD, 8))
    pool = _relu_pool_call(acc2.reshape(N_PAD, 128), y2, dinv8, b_g2_r, batch8)

    probs8, logits8 = _head_call(pool, cnt2, W_fc1.astype(f32), b_fc1_r,
                                 w_out8, b_out8)
    return (probs8[:, 0], logits8[:, 0])
